# baseline (speedup 1.0000x reference)
import jax
import jax.numpy as jnp
from jax import lax
from jax.experimental import pallas as pl
from jax.experimental.pallas import tpu as pltpu

_T = 2048
_D = 1024
_F = 4096
_E = 8
_K = 2
_R = 256
_NF = 2
_FH = _F // _NF
_MAXB = (_T * _K) // _R + _E
_P = _MAXB * _R
_NEG = -3.0e38


def _cumsum_lanes(x, n):
    lanes = lax.broadcasted_iota(jnp.int32, x.shape, x.ndim - 1)
    s = 1
    while s < n:
        x = x + jnp.where(lanes >= s, pltpu.roll(x, s, axis=x.ndim - 1), 0)
        s *= 2
    return x


def _cumsum_sublanes_excl(x, n):
    subs = lax.broadcasted_iota(jnp.int32, x.shape, 0)
    run = x
    s = 1
    while s < n:
        run = run + jnp.where(subs >= s, pltpu.roll(run, s, axis=0), 0)
        s *= 2
    return run - x


def _plan_kernel(lgt_ref, pos_ref, p_ref, be_ref, nb_ref):
    lgt = lgt_ref[...]
    isub = lax.broadcasted_iota(jnp.int32, (_E, _T), 0)

    m0 = jnp.max(lgt, axis=0, keepdims=True)
    e0 = jnp.min(jnp.where(lgt == m0, isub, _E), axis=0, keepdims=True)
    oh0 = (isub == e0)
    masked = jnp.where(oh0, _NEG, lgt)
    m1 = jnp.max(masked, axis=0, keepdims=True)
    e1 = jnp.min(jnp.where(masked == m1, isub, _E), axis=0, keepdims=True)
    oh1 = (isub == e1)

    ex = jnp.exp(m1 - m0)
    denom = 1.0 + ex
    p_ref[0:1, :] = 1.0 / denom
    p_ref[1:2, :] = ex / denom

    cs0 = _cumsum_lanes(oh0.astype(jnp.int32), _T)
    cs1 = _cumsum_lanes(oh1.astype(jnp.int32), _T)
    cnt0 = cs0[:, _T - 1:_T]
    counts = cnt0 + cs1[:, _T - 1:_T]
    nblk = (counts + (_R - 1)) // _R
    blk_start = _cumsum_sublanes_excl(nblk, _E)
    pad_start = blk_start * _R

    pos0 = jnp.sum(jnp.where(oh0, pad_start + cs0 - 1, 0), axis=0,
                   keepdims=True)
    pos1 = jnp.sum(jnp.where(oh1, pad_start + cnt0 + cs1 - 1, 0), axis=0,
                   keepdims=True)
    pos_ref[0:1, :] = pos0
    pos_ref[1:2, :] = pos1

    cnb = blk_start + nblk
    ilane = lax.broadcasted_iota(jnp.int32, (_E, 128), 1)
    be_raw = jnp.sum((cnb <= ilane).astype(jnp.int32), axis=0, keepdims=True)
    total = jnp.sum(nblk)
    last_e = jnp.sum((cnb <= total - 1).astype(jnp.int32), axis=0,
                     keepdims=True)[0:1, 0:1]
    be_ref[...] = jnp.where(ilane[0:1] < total, be_raw, last_e)
    nb_ref[...] = jnp.broadcast_to(total, (1, 1))


def _ffn_block_kernel(be_ref, nb_ref, x_ref, w1_ref, b1_ref, w2_ref, b2_ref,
                      o_ref, w1s, w2s):
    f = pl.program_id(0)
    i = pl.program_id(1)

    @pl.when(i < nb_ref[0])
    def _():
        new_w = (i == 0) | (be_ref[i] != be_ref[jnp.maximum(i - 1, 0)])

        @pl.when(new_w)
        def _():
            w1s[...] = w1_ref[0].astype(jnp.bfloat16)
            w2s[...] = w2_ref[0].astype(jnp.bfloat16)

        xb = x_ref[...]
        h = jnp.dot(xb, w1s[...], preferred_element_type=jnp.float32)
        h = h + b1_ref[0]
        h = 0.5 * h * (1.0 + jax.lax.erf(h * 0.7071067811865476))
        o = jnp.dot(h.astype(jnp.bfloat16), w2s[...],
                    preferred_element_type=jnp.float32)

        @pl.when(f == 0)
        def _():
            o_ref[0] = o + b2_ref[0]

        @pl.when(f != 0)
        def _():
            o_ref[0] = o


def kernel(x, W1, b1, W2, b2, Wr, br):
    bsz, seq, d = x.shape
    xf = x.reshape(-1, d)

    logits = xf @ Wr + br

    pos2, p2, be_row, nb = pl.pallas_call(
        _plan_kernel,
        out_shape=[
            jax.ShapeDtypeStruct((_K, _T), jnp.int32),
            jax.ShapeDtypeStruct((_K, _T), jnp.float32),
            jax.ShapeDtypeStruct((1, 128), jnp.int32),
            jax.ShapeDtypeStruct((1, 1), jnp.int32),
        ],
    )(logits.T)

    block_expert = be_row[0, :_MAXB]
    total_blocks = nb[0]

    pos_flat = pos2.reshape(-1)
    t_flat = jnp.tile(jnp.arange(_T, dtype=jnp.int32), _K)
    token_slot = jnp.zeros((_P,), jnp.int32).at[pos_flat].set(t_flat)
    x_sorted = jnp.take(xf, token_slot, axis=0).astype(jnp.bfloat16)

    grid_spec = pltpu.PrefetchScalarGridSpec(
        num_scalar_prefetch=2,
        grid=(_NF, _MAXB),
        in_specs=[
            pl.BlockSpec((_R, _D), lambda f, i, be, nb: (i, 0)),
            pl.BlockSpec((1, _D, _FH), lambda f, i, be, nb: (be[i], 0, f)),
            pl.BlockSpec((1, 1, _FH), lambda f, i, be, nb: (be[i], 0, f)),
            pl.BlockSpec((1, _FH, _D), lambda f, i, be, nb: (be[i], f, 0)),
            pl.BlockSpec((1, 1, _D), lambda f, i, be, nb: (be[i], 0, 0)),
        ],
        out_specs=pl.BlockSpec((1, _R, _D), lambda f, i, be, nb: (f, i, 0)),
        scratch_shapes=[
            pltpu.VMEM((_D, _FH), jnp.bfloat16),
            pltpu.VMEM((_FH, _D), jnp.bfloat16),
        ],
    )
    y = pl.pallas_call(
        _ffn_block_kernel,
        grid_spec=grid_spec,
        out_shape=jax.ShapeDtypeStruct((_NF, _P, _D), jnp.float32),
    )(block_expert, total_blocks, x_sorted, W1,
      b1.reshape(_E, 1, _F), W2, b2.reshape(_E, 1, _D))

    ys = y[0] + y[1]
    out = (p2[0][:, None] * jnp.take(ys, pos2[0], axis=0)
           + p2[1][:, None] * jnp.take(ys, pos2[1], axis=0))
    return out.reshape(bsz, seq, d)


# speedup vs baseline: 1.1521x; 1.1521x over previous
import functools

import jax
import jax.numpy as jnp
from jax import lax
from jax.experimental import pallas as pl
from jax.experimental.pallas import tpu as pltpu
from jax.experimental.pallas import tpu_sc as plsc

_T = 2048
_D = 1024
_F = 4096
_E = 8
_K = 2
_R = 256
_NF = 2
_FH = _F // _NF
_MAXB = (_T * _K) // _R + _E
_P = _MAXB * _R
_NEG = -3.0e38


def _cumsum_lanes(x, n):
    lanes = lax.broadcasted_iota(jnp.int32, x.shape, x.ndim - 1)
    s = 1
    while s < n:
        x = x + jnp.where(lanes >= s, pltpu.roll(x, s, axis=x.ndim - 1), 0)
        s *= 2
    return x


def _cumsum_sublanes_excl(x, n):
    subs = lax.broadcasted_iota(jnp.int32, x.shape, 0)
    run = x
    s = 1
    while s < n:
        run = run + jnp.where(subs >= s, pltpu.roll(run, s, axis=0), 0)
        s *= 2
    return run - x


def _plan_kernel(lgt_ref, pos_ref, p_ref, be_ref, nb_ref):
    lgt = lgt_ref[...]
    isub = lax.broadcasted_iota(jnp.int32, (_E, _T), 0)

    m0 = jnp.max(lgt, axis=0, keepdims=True)
    e0 = jnp.min(jnp.where(lgt == m0, isub, _E), axis=0, keepdims=True)
    oh0 = (isub == e0)
    masked = jnp.where(oh0, _NEG, lgt)
    m1 = jnp.max(masked, axis=0, keepdims=True)
    e1 = jnp.min(jnp.where(masked == m1, isub, _E), axis=0, keepdims=True)
    oh1 = (isub == e1)

    ex = jnp.exp(m1 - m0)
    denom = 1.0 + ex
    p_ref[0:1, :] = 1.0 / denom
    p_ref[1:2, :] = ex / denom

    cs0 = _cumsum_lanes(oh0.astype(jnp.int32), _T)
    cs1 = _cumsum_lanes(oh1.astype(jnp.int32), _T)
    cnt0 = cs0[:, _T - 1:_T]
    counts = cnt0 + cs1[:, _T - 1:_T]
    nblk = (counts + (_R - 1)) // _R
    blk_start = _cumsum_sublanes_excl(nblk, _E)
    pad_start = blk_start * _R

    pos0 = jnp.sum(jnp.where(oh0, pad_start + cs0 - 1, 0), axis=0,
                   keepdims=True)
    pos1 = jnp.sum(jnp.where(oh1, pad_start + cnt0 + cs1 - 1, 0), axis=0,
                   keepdims=True)
    pos_ref[0:1, :] = pos0
    pos_ref[1:2, :] = pos1

    cnb = blk_start + nblk
    ilane = lax.broadcasted_iota(jnp.int32, (_E, 128), 1)
    be_raw = jnp.sum((cnb <= ilane).astype(jnp.int32), axis=0, keepdims=True)
    total = jnp.sum(nblk)
    last_e = jnp.sum((cnb <= total - 1).astype(jnp.int32), axis=0,
                     keepdims=True)[0:1, 0:1]
    be_ref[...] = jnp.where(ilane[0:1] < total, be_raw, last_e)
    nb_ref[...] = jnp.broadcast_to(total, (1, 1))


_NW = 32
_JW = (_T * _K) // _NW
_CH = 64


def _make_dispatch_kernel():
    mesh = plsc.VectorSubcoreMesh(core_axis_name="c", subcore_axis_name="s")

    @functools.partial(
        pl.kernel, mesh=mesh,
        out_type=jax.ShapeDtypeStruct((_P, _D), jnp.float32),
        scratch_types=[
            pltpu.VMEM((_CH,), jnp.int32),
            pltpu.VMEM((_CH, _D), jnp.float32),
            pltpu.SemaphoreType.DMA,
        ],
    )
    def dispatch(pos_hbm, xf_hbm, xs_hbm, idx_v, rows_v, sem):
        wid = lax.axis_index("s") * 2 + lax.axis_index("c")
        t0 = (wid % (_T // _JW)) * _JW
        j0 = wid * _JW
        for cth in range(_JW // _CH):
            pltpu.sync_copy(pos_hbm.at[pl.ds(j0 + cth * _CH, _CH)], idx_v)
            pltpu.sync_copy(xf_hbm.at[pl.ds(t0 + cth * _CH, _CH)], rows_v)
            pltpu.async_copy(rows_v, xs_hbm.at[idx_v], sem).wait()

    return dispatch


def _ffn_block_kernel(be_ref, nb_ref, x_ref, w1_ref, b1_ref, w2_ref, b2_ref,
                      o_ref, w1s, w2s):
    f = pl.program_id(0)
    i = pl.program_id(1)

    @pl.when(i < nb_ref[0])
    def _():
        new_w = (i == 0) | (be_ref[i] != be_ref[jnp.maximum(i - 1, 0)])

        @pl.when(new_w)
        def _():
            w1s[...] = w1_ref[0].astype(jnp.bfloat16)
            w2s[...] = w2_ref[0].astype(jnp.bfloat16)

        xb = x_ref[...].astype(jnp.bfloat16)
        h = jnp.dot(xb, w1s[...], preferred_element_type=jnp.float32)
        h = h + b1_ref[0]
        h = 0.5 * h * (1.0 + jax.lax.erf(h * 0.7071067811865476))
        o = jnp.dot(h.astype(jnp.bfloat16), w2s[...],
                    preferred_element_type=jnp.float32)

        @pl.when(f == 0)
        def _():
            o_ref[0] = o + b2_ref[0]

        @pl.when(f != 0)
        def _():
            o_ref[0] = o


def kernel(x, W1, b1, W2, b2, Wr, br):
    bsz, seq, d = x.shape
    xf = x.reshape(-1, d)

    logits = xf @ Wr + br

    pos2, p2, be_row, nb = pl.pallas_call(
        _plan_kernel,
        out_shape=[
            jax.ShapeDtypeStruct((_K, _T), jnp.int32),
            jax.ShapeDtypeStruct((_K, _T), jnp.float32),
            jax.ShapeDtypeStruct((1, 128), jnp.int32),
            jax.ShapeDtypeStruct((1, 1), jnp.int32),
        ],
    )(logits.T)

    block_expert = be_row[0, :_MAXB]
    total_blocks = nb[0]

    pos_flat = pos2.reshape(-1)
    x_sorted = _make_dispatch_kernel()(pos_flat, xf)

    grid_spec = pltpu.PrefetchScalarGridSpec(
        num_scalar_prefetch=2,
        grid=(_NF, _MAXB),
        in_specs=[
            pl.BlockSpec((_R, _D), lambda f, i, be, nb: (i, 0)),
            pl.BlockSpec((1, _D, _FH), lambda f, i, be, nb: (be[i], 0, f)),
            pl.BlockSpec((1, 1, _FH), lambda f, i, be, nb: (be[i], 0, f)),
            pl.BlockSpec((1, _FH, _D), lambda f, i, be, nb: (be[i], f, 0)),
            pl.BlockSpec((1, 1, _D), lambda f, i, be, nb: (be[i], 0, 0)),
        ],
        out_specs=pl.BlockSpec((1, _R, _D), lambda f, i, be, nb: (f, i, 0)),
        scratch_shapes=[
            pltpu.VMEM((_D, _FH), jnp.bfloat16),
            pltpu.VMEM((_FH, _D), jnp.bfloat16),
        ],
    )
    y = pl.pallas_call(
        _ffn_block_kernel,
        grid_spec=grid_spec,
        out_shape=jax.ShapeDtypeStruct((_NF, _P, _D), jnp.float32),
    )(block_expert, total_blocks, x_sorted, W1,
      b1.reshape(_E, 1, _F), W2, b2.reshape(_E, 1, _D))

    ys = y[0] + y[1]
    out = (p2[0][:, None] * jnp.take(ys, pos2[0], axis=0)
           + p2[1][:, None] * jnp.take(ys, pos2[1], axis=0))
    return out.reshape(bsz, seq, d)


# speedup vs baseline: 1.1612x; 1.0079x over previous
import functools

import jax
import jax.numpy as jnp
from jax import lax
from jax.experimental import pallas as pl
from jax.experimental.pallas import tpu as pltpu
from jax.experimental.pallas import tpu_sc as plsc

_T = 2048
_D = 1024
_F = 4096
_E = 8
_K = 2
_R = 256
_NF = 2
_FH = _F // _NF
_MAXB = (_T * _K) // _R + _E
_P = _MAXB * _R
_NEG = -3.0e38


def _cumsum_lanes(x, n):
    lanes = lax.broadcasted_iota(jnp.int32, x.shape, x.ndim - 1)
    s = 1
    while s < n:
        x = x + jnp.where(lanes >= s, pltpu.roll(x, s, axis=x.ndim - 1), 0)
        s *= 2
    return x


def _cumsum_sublanes_excl(x, n):
    subs = lax.broadcasted_iota(jnp.int32, x.shape, 0)
    run = x
    s = 1
    while s < n:
        run = run + jnp.where(subs >= s, pltpu.roll(run, s, axis=0), 0)
        s *= 2
    return run - x


def _plan_kernel(lgt_ref, pos_ref, p_ref, be_ref, nb_ref):
    lgt = lgt_ref[...]
    isub = lax.broadcasted_iota(jnp.int32, (_E, _T), 0)

    m0 = jnp.max(lgt, axis=0, keepdims=True)
    e0 = jnp.min(jnp.where(lgt == m0, isub, _E), axis=0, keepdims=True)
    oh0 = (isub == e0)
    masked = jnp.where(oh0, _NEG, lgt)
    m1 = jnp.max(masked, axis=0, keepdims=True)
    e1 = jnp.min(jnp.where(masked == m1, isub, _E), axis=0, keepdims=True)
    oh1 = (isub == e1)

    ex = jnp.exp(m1 - m0)
    denom = 1.0 + ex
    p_ref[0:1, :] = 1.0 / denom
    p_ref[1:2, :] = ex / denom

    cs0 = _cumsum_lanes(oh0.astype(jnp.int32), _T)
    cs1 = _cumsum_lanes(oh1.astype(jnp.int32), _T)
    cnt0 = cs0[:, _T - 1:_T]
    counts = cnt0 + cs1[:, _T - 1:_T]
    nblk = (counts + (_R - 1)) // _R
    blk_start = _cumsum_sublanes_excl(nblk, _E)
    pad_start = blk_start * _R

    pos0 = jnp.sum(jnp.where(oh0, pad_start + cs0 - 1, 0), axis=0,
                   keepdims=True)
    pos1 = jnp.sum(jnp.where(oh1, pad_start + cnt0 + cs1 - 1, 0), axis=0,
                   keepdims=True)
    pos_ref[0:1, :] = pos0
    pos_ref[1:2, :] = pos1

    cnb = blk_start + nblk
    ilane = lax.broadcasted_iota(jnp.int32, (_E, 128), 1)
    be_raw = jnp.sum((cnb <= ilane).astype(jnp.int32), axis=0, keepdims=True)
    total = jnp.sum(nblk)
    last_e = jnp.sum((cnb <= total - 1).astype(jnp.int32), axis=0,
                     keepdims=True)[0:1, 0:1]
    be_ref[...] = jnp.where(ilane[0:1] < total, be_raw, last_e)
    nb_ref[...] = jnp.broadcast_to(total, (1, 1))


_NW = 32
_JW = (_T * _K) // _NW
_CH = 64


def _make_dispatch_kernel():
    mesh = plsc.VectorSubcoreMesh(core_axis_name="c", subcore_axis_name="s")

    @functools.partial(
        pl.kernel, mesh=mesh,
        out_type=jax.ShapeDtypeStruct((_P, _D), jnp.float32),
        scratch_types=[
            pltpu.VMEM((_CH,), jnp.int32),
            pltpu.VMEM((_CH, _D), jnp.float32),
            pltpu.SemaphoreType.DMA,
        ],
    )
    def dispatch(pos_hbm, xf_hbm, xs_hbm, idx_v, rows_v, sem):
        wid = lax.axis_index("s") * 2 + lax.axis_index("c")
        t0 = (wid % (_T // _JW)) * _JW
        j0 = wid * _JW
        for cth in range(_JW // _CH):
            pltpu.sync_copy(pos_hbm.at[pl.ds(j0 + cth * _CH, _CH)], idx_v)
            pltpu.sync_copy(xf_hbm.at[pl.ds(t0 + cth * _CH, _CH)], rows_v)
            pltpu.async_copy(rows_v, xs_hbm.at[idx_v], sem).wait()

    return dispatch


_TW = _T // _NW
_CT = 16


def _make_combine_kernel():
    mesh = plsc.VectorSubcoreMesh(core_axis_name="c", subcore_axis_name="s")
    nsteps = (_TW // _CT) * 2

    @functools.partial(
        pl.kernel, mesh=mesh,
        out_type=jax.ShapeDtypeStruct((_T, _D), jnp.float32),
        scratch_types=[
            pltpu.VMEM((_TW,), jnp.int32),
            pltpu.VMEM((_TW,), jnp.int32),
            pltpu.VMEM((_TW, 16), jnp.float32),
            pltpu.VMEM((_TW, 16), jnp.float32),
            pltpu.VMEM((32,), jnp.int32),
            pltpu.VMEM((32,), jnp.int32),
            pltpu.VMEM((2 * _CT, _D), jnp.float32),
            pltpu.VMEM((2 * _CT, _D), jnp.float32),
            pltpu.VMEM((_CT, _D), jnp.float32),
            pltpu.SemaphoreType.DMA,
            pltpu.SemaphoreType.DMA,
        ],
    )
    def combine(pos_hbm, pb_hbm, y_hbm, out_hbm, pos0_w, pos1_w, pb0_w,
                pb1_w, idx_a, idx_b, rows_a, rows_b, out_c, sem_a, sem_b):
        wid = lax.axis_index("s") * 2 + lax.axis_index("c")
        t0 = wid * _TW
        pltpu.sync_copy(pos_hbm.at[0, pl.ds(t0, _TW)], pos0_w)
        pltpu.sync_copy(pos_hbm.at[1, pl.ds(t0, _TW)], pos1_w)
        pltpu.sync_copy(pb_hbm.at[0, pl.ds(t0, _TW)], pb0_w)
        pltpu.sync_copy(pb_hbm.at[1, pl.ds(t0, _TW)], pb1_w)

        def fire(s):
            c, h = s // 2, s % 2
            posw = pos0_w if h == 0 else pos1_w
            idxv = idx_a if s % 2 == 0 else idx_b
            rowsv = rows_a if s % 2 == 0 else rows_b
            semv = sem_a if s % 2 == 0 else sem_b
            q = posw[pl.ds(c * _CT, _CT)]
            idxv[pl.ds(0, _CT)] = q
            idxv[pl.ds(_CT, _CT)] = q + _P
            return pltpu.async_copy(y_hbm.at[idxv], rowsv, semv)

        def compute(s):
            c, h = s // 2, s % 2
            pbw = pb0_w if h == 0 else pb1_w
            rowsv = rows_a if s % 2 == 0 else rows_b
            for i in range(_CT):
                pb = pbw[c * _CT + i, pl.ds(0, 16)]

                def body(j, acc, i=i, pb=pb, rowsv=rowsv, h=h):
                    sl = pl.ds(j * 16, 16)
                    a = rowsv[i, sl] + rowsv[_CT + i, sl]
                    if h == 0:
                        out_c[i, sl] = pb * a
                    else:
                        out_c[i, sl] = out_c[i, sl] + pb * a
                    return acc

                lax.fori_loop(0, _D // 16, body, 0)

        pending = fire(0)
        for s in range(nsteps):
            nxt = fire(s + 1) if s + 1 < nsteps else None
            pending.wait()
            compute(s)
            if s % 2 == 1:
                pltpu.sync_copy(
                    out_c, out_hbm.at[pl.ds(t0 + (s // 2) * _CT, _CT)])
            pending = nxt

    return combine


def _ffn_block_kernel(be_ref, nb_ref, x_ref, w1_ref, b1_ref, w2_ref, b2_ref,
                      o_ref, w1s, w2s):
    f = pl.program_id(0)
    i = pl.program_id(1)

    @pl.when(i < nb_ref[0])
    def _():
        new_w = (i == 0) | (be_ref[i] != be_ref[jnp.maximum(i - 1, 0)])

        @pl.when(new_w)
        def _():
            w1s[...] = w1_ref[0].astype(jnp.bfloat16)
            w2s[...] = w2_ref[0].astype(jnp.bfloat16)

        xb = x_ref[...].astype(jnp.bfloat16)
        h = jnp.dot(xb, w1s[...], preferred_element_type=jnp.float32)
        h = h + b1_ref[0]
        h = 0.5 * h * (1.0 + jax.lax.erf(h * 0.7071067811865476))
        o = jnp.dot(h.astype(jnp.bfloat16), w2s[...],
                    preferred_element_type=jnp.float32)

        @pl.when(f == 0)
        def _():
            o_ref[0] = o + b2_ref[0]

        @pl.when(f != 0)
        def _():
            o_ref[0] = o


def kernel(x, W1, b1, W2, b2, Wr, br):
    bsz, seq, d = x.shape
    xf = x.reshape(-1, d)

    logits = xf @ Wr + br

    pos2, p2, be_row, nb = pl.pallas_call(
        _plan_kernel,
        out_shape=[
            jax.ShapeDtypeStruct((_K, _T), jnp.int32),
            jax.ShapeDtypeStruct((_K, _T), jnp.float32),
            jax.ShapeDtypeStruct((1, 128), jnp.int32),
            jax.ShapeDtypeStruct((1, 1), jnp.int32),
        ],
    )(logits.T)

    block_expert = be_row[0, :_MAXB]
    total_blocks = nb[0]

    pos_flat = pos2.reshape(-1)
    x_sorted = _make_dispatch_kernel()(pos_flat, xf)

    grid_spec = pltpu.PrefetchScalarGridSpec(
        num_scalar_prefetch=2,
        grid=(_NF, _MAXB),
        in_specs=[
            pl.BlockSpec((_R, _D), lambda f, i, be, nb: (i, 0)),
            pl.BlockSpec((1, _D, _FH), lambda f, i, be, nb: (be[i], 0, f)),
            pl.BlockSpec((1, 1, _FH), lambda f, i, be, nb: (be[i], 0, f)),
            pl.BlockSpec((1, _FH, _D), lambda f, i, be, nb: (be[i], f, 0)),
            pl.BlockSpec((1, 1, _D), lambda f, i, be, nb: (be[i], 0, 0)),
        ],
        out_specs=pl.BlockSpec((1, _R, _D), lambda f, i, be, nb: (f, i, 0)),
        scratch_shapes=[
            pltpu.VMEM((_D, _FH), jnp.bfloat16),
            pltpu.VMEM((_FH, _D), jnp.bfloat16),
        ],
    )
    y = pl.pallas_call(
        _ffn_block_kernel,
        grid_spec=grid_spec,
        out_shape=jax.ShapeDtypeStruct((_NF, _P, _D), jnp.float32),
    )(block_expert, total_blocks, x_sorted, W1,
      b1.reshape(_E, 1, _F), W2, b2.reshape(_E, 1, _D))

    pbb = jnp.broadcast_to(p2[:, :, None], (_K, _T, 16))
    out = _make_combine_kernel()(pos2, pbb, y.reshape(_NF * _P, _D))
    return out.reshape(bsz, seq, d)


# speedup vs baseline: 1.2374x; 1.0655x over previous
import functools

import jax
import jax.numpy as jnp
from jax import lax
from jax.experimental import pallas as pl
from jax.experimental.pallas import tpu as pltpu
from jax.experimental.pallas import tpu_sc as plsc

_T = 2048
_D = 1024
_F = 4096
_E = 8
_K = 2
_R = 256
_NF = 2
_FH = _F // _NF
_MAXB = (_T * _K) // _R + _E
_P = _MAXB * _R
_NEG = -3.0e38


def _cumsum_lanes(x, n):
    lanes = lax.broadcasted_iota(jnp.int32, x.shape, x.ndim - 1)
    s = 1
    while s < n:
        x = x + jnp.where(lanes >= s, pltpu.roll(x, s, axis=x.ndim - 1), 0)
        s *= 2
    return x


def _cumsum_sublanes_excl(x, n):
    subs = lax.broadcasted_iota(jnp.int32, x.shape, 0)
    run = x
    s = 1
    while s < n:
        run = run + jnp.where(subs >= s, pltpu.roll(run, s, axis=0), 0)
        s *= 2
    return run - x


def _plan_kernel(lgt_ref, pos_ref, p_ref, be_ref, nb_ref):
    lgt = lgt_ref[...]
    isub = lax.broadcasted_iota(jnp.int32, (_E, _T), 0)

    m0 = jnp.max(lgt, axis=0, keepdims=True)
    e0 = jnp.min(jnp.where(lgt == m0, isub, _E), axis=0, keepdims=True)
    oh0 = (isub == e0)
    masked = jnp.where(oh0, _NEG, lgt)
    m1 = jnp.max(masked, axis=0, keepdims=True)
    e1 = jnp.min(jnp.where(masked == m1, isub, _E), axis=0, keepdims=True)
    oh1 = (isub == e1)

    ex = jnp.exp(m1 - m0)
    denom = 1.0 + ex
    p_ref[0:1, :] = 1.0 / denom
    p_ref[1:2, :] = ex / denom

    cs0 = _cumsum_lanes(oh0.astype(jnp.int32), _T)
    cs1 = _cumsum_lanes(oh1.astype(jnp.int32), _T)
    cnt0 = cs0[:, _T - 1:_T]
    counts = cnt0 + cs1[:, _T - 1:_T]
    nblk = (counts + (_R - 1)) // _R
    blk_start = _cumsum_sublanes_excl(nblk, _E)
    pad_start = blk_start * _R

    pos0 = jnp.sum(jnp.where(oh0, pad_start + cs0 - 1, 0), axis=0,
                   keepdims=True)
    pos1 = jnp.sum(jnp.where(oh1, pad_start + cnt0 + cs1 - 1, 0), axis=0,
                   keepdims=True)
    pos_ref[0:1, :] = pos0
    pos_ref[1:2, :] = pos1

    cnb = blk_start + nblk
    ilane = lax.broadcasted_iota(jnp.int32, (_E, 128), 1)
    be_raw = jnp.sum((cnb <= ilane).astype(jnp.int32), axis=0, keepdims=True)
    total = jnp.sum(nblk)
    last_e = jnp.sum((cnb <= total - 1).astype(jnp.int32), axis=0,
                     keepdims=True)[0:1, 0:1]
    be_ref[...] = jnp.where(ilane[0:1] < total, be_raw, last_e)
    nb_ref[...] = jnp.broadcast_to(total, (1, 1))


_NW = 32
_JW = (_T * _K) // _NW
_CH = 64


def _make_dispatch_kernel():
    mesh = plsc.VectorSubcoreMesh(core_axis_name="c", subcore_axis_name="s")

    @functools.partial(
        pl.kernel, mesh=mesh,
        out_type=jax.ShapeDtypeStruct((_P, _D), jnp.float32),
        scratch_types=[
            pltpu.VMEM((_CH,), jnp.int32),
            pltpu.VMEM((_CH, _D), jnp.float32),
            pltpu.SemaphoreType.DMA,
        ],
    )
    def dispatch(pos_hbm, xf_hbm, xs_hbm, idx_v, rows_v, sem):
        wid = lax.axis_index("s") * 2 + lax.axis_index("c")
        t0 = (wid % (_T // _JW)) * _JW
        j0 = wid * _JW
        for cth in range(_JW // _CH):
            pltpu.sync_copy(pos_hbm.at[pl.ds(j0 + cth * _CH, _CH)], idx_v)
            pltpu.sync_copy(xf_hbm.at[pl.ds(t0 + cth * _CH, _CH)], rows_v)
            pltpu.async_copy(rows_v, xs_hbm.at[idx_v], sem).wait()

    return dispatch


_TW = _T // _NW
_CT = 16


def _make_combine_kernel():
    mesh = plsc.VectorSubcoreMesh(core_axis_name="c", subcore_axis_name="s")
    nsteps = (_TW // _CT) * 2

    @functools.partial(
        pl.kernel, mesh=mesh,
        out_type=jax.ShapeDtypeStruct((_T, _D), jnp.float32),
        scratch_types=[
            pltpu.VMEM((_TW,), jnp.int32),
            pltpu.VMEM((_TW,), jnp.int32),
            pltpu.VMEM((_TW, 16), jnp.float32),
            pltpu.VMEM((_TW, 16), jnp.float32),
            pltpu.VMEM((32,), jnp.int32),
            pltpu.VMEM((32,), jnp.int32),
            pltpu.VMEM((2 * _CT, _D), jnp.float32),
            pltpu.VMEM((2 * _CT, _D), jnp.float32),
            pltpu.VMEM((_CT, _D), jnp.float32),
            pltpu.SemaphoreType.DMA,
            pltpu.SemaphoreType.DMA,
        ],
    )
    def combine(pos_hbm, pb_hbm, y_hbm, out_hbm, pos0_w, pos1_w, pb0_w,
                pb1_w, idx_a, idx_b, rows_a, rows_b, out_c, sem_a, sem_b):
        wid = lax.axis_index("s") * 2 + lax.axis_index("c")
        t0 = wid * _TW
        pltpu.sync_copy(pos_hbm.at[0, pl.ds(t0, _TW)], pos0_w)
        pltpu.sync_copy(pos_hbm.at[1, pl.ds(t0, _TW)], pos1_w)
        pltpu.sync_copy(pb_hbm.at[0, pl.ds(t0, _TW)], pb0_w)
        pltpu.sync_copy(pb_hbm.at[1, pl.ds(t0, _TW)], pb1_w)

        def fire(s):
            c, h = s // 2, s % 2
            posw = pos0_w if h == 0 else pos1_w
            idxv = idx_a if s % 2 == 0 else idx_b
            rowsv = rows_a if s % 2 == 0 else rows_b
            semv = sem_a if s % 2 == 0 else sem_b
            q = posw[pl.ds(c * _CT, _CT)]
            idxv[pl.ds(0, _CT)] = q
            idxv[pl.ds(_CT, _CT)] = q + _P
            return pltpu.async_copy(y_hbm.at[idxv], rowsv, semv)

        def compute(s):
            c, h = s // 2, s % 2
            pbw = pb0_w if h == 0 else pb1_w
            rowsv = rows_a if s % 2 == 0 else rows_b
            pbs = [pbw[c * _CT + i, pl.ds(0, 16)] for i in range(_CT)]

            def body(j, acc):
                sl = pl.ds(j * 16, 16)
                for i in range(_CT):
                    a = rowsv[i, sl] + rowsv[_CT + i, sl]
                    if h == 0:
                        out_c[i, sl] = pbs[i] * a
                    else:
                        out_c[i, sl] = out_c[i, sl] + pbs[i] * a
                return acc

            lax.fori_loop(0, _D // 16, body, 0)

        pending = fire(0)
        for s in range(nsteps):
            nxt = fire(s + 1) if s + 1 < nsteps else None
            pending.wait()
            compute(s)
            if s % 2 == 1:
                pltpu.sync_copy(
                    out_c, out_hbm.at[pl.ds(t0 + (s // 2) * _CT, _CT)])
            pending = nxt

    return combine


def _ffn_block_kernel(be_ref, nb_ref, x_ref, w1_ref, b1_ref, w2_ref, b2_ref,
                      o_ref, w1s, w2s):
    f = pl.program_id(0)
    i = pl.program_id(1)

    @pl.when(i < nb_ref[0])
    def _():
        new_w = (i == 0) | (be_ref[i] != be_ref[jnp.maximum(i - 1, 0)])

        @pl.when(new_w)
        def _():
            w1s[...] = w1_ref[0].astype(jnp.bfloat16)
            w2s[...] = w2_ref[0].astype(jnp.bfloat16)

        xb = x_ref[...].astype(jnp.bfloat16)
        h = jnp.dot(xb, w1s[...], preferred_element_type=jnp.float32)
        h = h + b1_ref[0]
        h = 0.5 * h * (1.0 + jax.lax.erf(h * 0.7071067811865476))
        o = jnp.dot(h.astype(jnp.bfloat16), w2s[...],
                    preferred_element_type=jnp.float32)

        @pl.when(f == 0)
        def _():
            o_ref[0] = o + b2_ref[0]

        @pl.when(f != 0)
        def _():
            o_ref[0] = o


def kernel(x, W1, b1, W2, b2, Wr, br):
    bsz, seq, d = x.shape
    xf = x.reshape(-1, d)

    logits = xf @ Wr + br

    pos2, p2, be_row, nb = pl.pallas_call(
        _plan_kernel,
        out_shape=[
            jax.ShapeDtypeStruct((_K, _T), jnp.int32),
            jax.ShapeDtypeStruct((_K, _T), jnp.float32),
            jax.ShapeDtypeStruct((1, 128), jnp.int32),
            jax.ShapeDtypeStruct((1, 1), jnp.int32),
        ],
    )(logits.T)

    block_expert = be_row[0, :_MAXB]
    total_blocks = nb[0]

    pos_flat = pos2.reshape(-1)
    x_sorted = _make_dispatch_kernel()(pos_flat, xf)

    grid_spec = pltpu.PrefetchScalarGridSpec(
        num_scalar_prefetch=2,
        grid=(_NF, _MAXB),
        in_specs=[
            pl.BlockSpec((_R, _D), lambda f, i, be, nb: (i, 0)),
            pl.BlockSpec((1, _D, _FH), lambda f, i, be, nb: (be[i], 0, f)),
            pl.BlockSpec((1, 1, _FH), lambda f, i, be, nb: (be[i], 0, f)),
            pl.BlockSpec((1, _FH, _D), lambda f, i, be, nb: (be[i], f, 0)),
            pl.BlockSpec((1, 1, _D), lambda f, i, be, nb: (be[i], 0, 0)),
        ],
        out_specs=pl.BlockSpec((1, _R, _D), lambda f, i, be, nb: (f, i, 0)),
        scratch_shapes=[
            pltpu.VMEM((_D, _FH), jnp.bfloat16),
            pltpu.VMEM((_FH, _D), jnp.bfloat16),
        ],
    )
    y = pl.pallas_call(
        _ffn_block_kernel,
        grid_spec=grid_spec,
        out_shape=jax.ShapeDtypeStruct((_NF, _P, _D), jnp.float32),
    )(block_expert, total_blocks, x_sorted, W1,
      b1.reshape(_E, 1, _F), W2, b2.reshape(_E, 1, _D))

    pbb = jnp.broadcast_to(p2[:, :, None], (_K, _T, 16))
    out = _make_combine_kernel()(pos2, pbb, y.reshape(_NF * _P, _D))
    return out.reshape(bsz, seq, d)


# speedup vs baseline: 1.2506x; 1.0107x over previous
import functools

import jax
import jax.numpy as jnp
from jax import lax
from jax.experimental import pallas as pl
from jax.experimental.pallas import tpu as pltpu
from jax.experimental.pallas import tpu_sc as plsc

_T = 2048
_D = 1024
_F = 4096
_E = 8
_K = 2
_R = 256
_NF = 2
_FH = _F // _NF
_MAXB = (_T * _K) // _R + _E
_P = _MAXB * _R
_NEG = -3.0e38


def _cumsum_lanes(x, n):
    lanes = lax.broadcasted_iota(jnp.int32, x.shape, x.ndim - 1)
    s = 1
    while s < n:
        x = x + jnp.where(lanes >= s, pltpu.roll(x, s, axis=x.ndim - 1), 0)
        s *= 2
    return x


def _cumsum_sublanes_excl(x, n):
    subs = lax.broadcasted_iota(jnp.int32, x.shape, 0)
    run = x
    s = 1
    while s < n:
        run = run + jnp.where(subs >= s, pltpu.roll(run, s, axis=0), 0)
        s *= 2
    return run - x


def _plan_kernel(lgt_ref, pos_ref, p_ref, be_ref, nb_ref):
    lgt = lgt_ref[...]
    isub = lax.broadcasted_iota(jnp.int32, (_E, _T), 0)

    m0 = jnp.max(lgt, axis=0, keepdims=True)
    e0 = jnp.min(jnp.where(lgt == m0, isub, _E), axis=0, keepdims=True)
    oh0 = (isub == e0)
    masked = jnp.where(oh0, _NEG, lgt)
    m1 = jnp.max(masked, axis=0, keepdims=True)
    e1 = jnp.min(jnp.where(masked == m1, isub, _E), axis=0, keepdims=True)
    oh1 = (isub == e1)

    ex = jnp.exp(m1 - m0)
    denom = 1.0 + ex
    p_ref[0:1, :] = 1.0 / denom
    p_ref[1:2, :] = ex / denom

    cs0 = _cumsum_lanes(oh0.astype(jnp.int32), _T)
    cs1 = _cumsum_lanes(oh1.astype(jnp.int32), _T)
    cnt0 = cs0[:, _T - 1:_T]
    counts = cnt0 + cs1[:, _T - 1:_T]
    nblk = (counts + (_R - 1)) // _R
    blk_start = _cumsum_sublanes_excl(nblk, _E)
    pad_start = blk_start * _R

    pos0 = jnp.sum(jnp.where(oh0, pad_start + cs0 - 1, 0), axis=0,
                   keepdims=True)
    pos1 = jnp.sum(jnp.where(oh1, pad_start + cnt0 + cs1 - 1, 0), axis=0,
                   keepdims=True)
    pos_ref[0:1, :] = pos0
    pos_ref[1:2, :] = pos1

    cnb = blk_start + nblk
    ilane = lax.broadcasted_iota(jnp.int32, (_E, 128), 1)
    be_raw = jnp.sum((cnb <= ilane).astype(jnp.int32), axis=0, keepdims=True)
    total = jnp.sum(nblk)
    last_e = jnp.sum((cnb <= total - 1).astype(jnp.int32), axis=0,
                     keepdims=True)[0:1, 0:1]
    be_ref[...] = jnp.where(ilane[0:1] < total, be_raw, last_e)
    nb_ref[...] = jnp.broadcast_to(total, (1, 1))


_NW = 32
_JW = (_T * _K) // _NW
_CH = 64


def _make_dispatch_kernel():
    mesh = plsc.VectorSubcoreMesh(core_axis_name="c", subcore_axis_name="s")

    @functools.partial(
        pl.kernel, mesh=mesh,
        out_type=jax.ShapeDtypeStruct((_P, _D), jnp.float32),
        scratch_types=[
            pltpu.VMEM((_CH,), jnp.int32),
            pltpu.VMEM((_CH, _D), jnp.float32),
            pltpu.SemaphoreType.DMA,
        ],
    )
    def dispatch(pos_hbm, xf_hbm, xs_hbm, idx_v, rows_v, sem):
        wid = lax.axis_index("s") * 2 + lax.axis_index("c")
        t0 = (wid % (_T // _JW)) * _JW
        j0 = wid * _JW
        for cth in range(_JW // _CH):
            pltpu.sync_copy(pos_hbm.at[pl.ds(j0 + cth * _CH, _CH)], idx_v)
            pltpu.sync_copy(xf_hbm.at[pl.ds(t0 + cth * _CH, _CH)], rows_v)
            pltpu.async_copy(rows_v, xs_hbm.at[idx_v], sem).wait()

    return dispatch


_TW = _T // _NW
_CT = 16


def _make_combine_kernel():
    mesh = plsc.VectorSubcoreMesh(core_axis_name="c", subcore_axis_name="s")
    nsteps = (_TW // _CT) * 2

    @functools.partial(
        pl.kernel, mesh=mesh,
        out_type=jax.ShapeDtypeStruct((_T, _D), jnp.float32),
        scratch_types=[
            pltpu.VMEM((_TW,), jnp.int32),
            pltpu.VMEM((_TW,), jnp.int32),
            pltpu.VMEM((_TW, 16), jnp.float32),
            pltpu.VMEM((_TW, 16), jnp.float32),
            pltpu.VMEM((32,), jnp.int32),
            pltpu.VMEM((32,), jnp.int32),
            pltpu.VMEM((2 * _CT, _D), jnp.float32),
            pltpu.VMEM((2 * _CT, _D), jnp.float32),
            pltpu.VMEM((_CT, _D), jnp.float32),
            pltpu.SemaphoreType.DMA,
            pltpu.SemaphoreType.DMA,
        ],
    )
    def combine(pos_hbm, pb_hbm, y_hbm, out_hbm, pos0_w, pos1_w, pb0_w,
                pb1_w, idx_a, idx_b, rows_a, rows_b, out_c, sem_a, sem_b):
        wid = lax.axis_index("s") * 2 + lax.axis_index("c")
        t0 = wid * _TW
        pltpu.sync_copy(pos_hbm.at[0, pl.ds(t0, _TW)], pos0_w)
        pltpu.sync_copy(pos_hbm.at[1, pl.ds(t0, _TW)], pos1_w)
        pltpu.sync_copy(pb_hbm.at[0, pl.ds(t0, _TW)], pb0_w)
        pltpu.sync_copy(pb_hbm.at[1, pl.ds(t0, _TW)], pb1_w)

        def fire(s):
            c, h = s // 2, s % 2
            posw = pos0_w if h == 0 else pos1_w
            idxv = idx_a if s % 2 == 0 else idx_b
            rowsv = rows_a if s % 2 == 0 else rows_b
            semv = sem_a if s % 2 == 0 else sem_b
            q = posw[pl.ds(c * _CT, _CT)]
            idxv[pl.ds(0, _CT)] = q
            idxv[pl.ds(_CT, _CT)] = q + _P
            return pltpu.async_copy(y_hbm.at[idxv], rowsv, semv)

        def compute(s):
            c, h = s // 2, s % 2
            pbw = pb0_w if h == 0 else pb1_w
            rowsv = rows_a if s % 2 == 0 else rows_b
            pbs = [pbw[c * _CT + i, pl.ds(0, 16)] for i in range(_CT)]

            def body(j, acc):
                sl = pl.ds(j * 16, 16)
                for i in range(_CT):
                    a = rowsv[i, sl] + rowsv[_CT + i, sl]
                    if h == 0:
                        out_c[i, sl] = pbs[i] * a
                    else:
                        out_c[i, sl] = out_c[i, sl] + pbs[i] * a
                return acc

            lax.fori_loop(0, _D // 16, body, 0)

        pending = fire(0)
        for s in range(nsteps):
            nxt = fire(s + 1) if s + 1 < nsteps else None
            pending.wait()
            compute(s)
            if s % 2 == 1:
                pltpu.sync_copy(
                    out_c, out_hbm.at[pl.ds(t0 + (s // 2) * _CT, _CT)])
            pending = nxt

    return combine


def _ffn_block_kernel(be_ref, nb_ref, x_ref, w1_ref, b1_ref, w2_ref, b2_ref,
                      o_ref, w1s, w2s):
    f = pl.program_id(0)
    i = pl.program_id(1)

    @pl.when(i < nb_ref[0])
    def _():
        new_w = (i == 0) | (be_ref[i] != be_ref[jnp.maximum(i - 1, 0)])

        @pl.when(new_w)
        def _():
            w1s[...] = w1_ref[0].astype(jnp.bfloat16)
            w2s[...] = w2_ref[0].astype(jnp.bfloat16)

        xb = x_ref[...].astype(jnp.bfloat16)
        h = jnp.dot(xb, w1s[...], preferred_element_type=jnp.float32)
        h = h + b1_ref[0]
        h = 0.5 * h * (1.0 + jax.lax.erf(h * 0.7071067811865476))
        o = jnp.dot(h.astype(jnp.bfloat16), w2s[...],
                    preferred_element_type=jnp.float32)

        @pl.when(f == 0)
        def _():
            o_ref[0] = o + b2_ref[0]

        @pl.when(f != 0)
        def _():
            o_ref[0] = o


def kernel(x, W1, b1, W2, b2, Wr, br):
    bsz, seq, d = x.shape
    xf = x.reshape(-1, d)

    logits = xf @ Wr + br

    pos2, p2, be_row, nb = pl.pallas_call(
        _plan_kernel,
        out_shape=[
            jax.ShapeDtypeStruct((_K, _T), jnp.int32),
            jax.ShapeDtypeStruct((_K, _T), jnp.float32),
            jax.ShapeDtypeStruct((1, 128), jnp.int32),
            jax.ShapeDtypeStruct((1, 1), jnp.int32),
        ],
    )(logits.T)

    block_expert = be_row[0, :_MAXB]
    total_blocks = nb[0]

    pos_flat = pos2.reshape(-1)
    x_sorted = _make_dispatch_kernel()(pos_flat, xf)

    grid_spec = pltpu.PrefetchScalarGridSpec(
        num_scalar_prefetch=2,
        grid=(_NF, _MAXB),
        in_specs=[
            pl.BlockSpec((_R, _D),
                         lambda f, i, be, nb: (jnp.where(i < nb[0], i, 0), 0)),
            pl.BlockSpec((1, _D, _FH), lambda f, i, be, nb: (be[i], 0, f)),
            pl.BlockSpec((1, 1, _FH), lambda f, i, be, nb: (be[i], 0, f)),
            pl.BlockSpec((1, _FH, _D), lambda f, i, be, nb: (be[i], f, 0)),
            pl.BlockSpec((1, 1, _D), lambda f, i, be, nb: (be[i], 0, 0)),
        ],
        out_specs=pl.BlockSpec((1, _R, _D), lambda f, i, be, nb: (f, i, 0)),
        scratch_shapes=[
            pltpu.VMEM((_D, _FH), jnp.bfloat16),
            pltpu.VMEM((_FH, _D), jnp.bfloat16),
        ],
    )
    y = pl.pallas_call(
        _ffn_block_kernel,
        grid_spec=grid_spec,
        out_shape=jax.ShapeDtypeStruct((_NF, _P, _D), jnp.float32),
    )(block_expert, total_blocks, x_sorted, W1,
      b1.reshape(_E, 1, _F), W2, b2.reshape(_E, 1, _D))

    pbb = jnp.broadcast_to(p2[:, :, None], (_K, _T, 16))
    out = _make_combine_kernel()(pos2, pbb, y.reshape(_NF * _P, _D))
    return out.reshape(bsz, seq, d)


# speedup vs baseline: 1.3938x; 1.1145x over previous
import functools

import jax
import jax.numpy as jnp
from jax import lax
from jax.experimental import pallas as pl
from jax.experimental.pallas import tpu as pltpu
from jax.experimental.pallas import tpu_sc as plsc

_T = 2048
_D = 1024
_F = 4096
_E = 8
_K = 2
_R = 256
_NF = 2
_FH = _F // _NF
_MAXB = (_T * _K) // _R + _E
_P = _MAXB * _R
_NEG = -3.0e38


def _cumsum_lanes(x, n):
    lanes = lax.broadcasted_iota(jnp.int32, x.shape, x.ndim - 1)
    s = 1
    while s < n:
        x = x + jnp.where(lanes >= s, pltpu.roll(x, s, axis=x.ndim - 1), 0)
        s *= 2
    return x


def _cumsum_sublanes_excl(x, n):
    subs = lax.broadcasted_iota(jnp.int32, x.shape, 0)
    run = x
    s = 1
    while s < n:
        run = run + jnp.where(subs >= s, pltpu.roll(run, s, axis=0), 0)
        s *= 2
    return run - x


def _plan_kernel(lgt_ref, pos_ref, p_ref, be_ref, nb_ref):
    lgt = lgt_ref[...]
    isub = lax.broadcasted_iota(jnp.int32, (_E, _T), 0)

    m0 = jnp.max(lgt, axis=0, keepdims=True)
    e0 = jnp.min(jnp.where(lgt == m0, isub, _E), axis=0, keepdims=True)
    oh0 = (isub == e0)
    masked = jnp.where(oh0, _NEG, lgt)
    m1 = jnp.max(masked, axis=0, keepdims=True)
    e1 = jnp.min(jnp.where(masked == m1, isub, _E), axis=0, keepdims=True)
    oh1 = (isub == e1)

    ex = jnp.exp(m1 - m0)
    denom = 1.0 + ex
    p_ref[0:1, :] = 1.0 / denom
    p_ref[1:2, :] = ex / denom

    cs0 = _cumsum_lanes(oh0.astype(jnp.int32), _T)
    cs1 = _cumsum_lanes(oh1.astype(jnp.int32), _T)
    cnt0 = cs0[:, _T - 1:_T]
    counts = cnt0 + cs1[:, _T - 1:_T]
    nblk = (counts + (_R - 1)) // _R
    blk_start = _cumsum_sublanes_excl(nblk, _E)
    pad_start = blk_start * _R

    pos0 = jnp.sum(jnp.where(oh0, pad_start + cs0 - 1, 0), axis=0,
                   keepdims=True)
    pos1 = jnp.sum(jnp.where(oh1, pad_start + cnt0 + cs1 - 1, 0), axis=0,
                   keepdims=True)
    pos_ref[0:1, :] = pos0
    pos_ref[1:2, :] = pos1

    cnb = blk_start + nblk
    ilane = lax.broadcasted_iota(jnp.int32, (_E, 128), 1)
    be_raw = jnp.sum((cnb <= ilane).astype(jnp.int32), axis=0, keepdims=True)
    total = jnp.sum(nblk)
    last_e = jnp.sum((cnb <= total - 1).astype(jnp.int32), axis=0,
                     keepdims=True)[0:1, 0:1]
    be_ref[...] = jnp.where(ilane[0:1] < total, be_raw, last_e)
    nb_ref[...] = jnp.broadcast_to(total, (1, 1))


_NW = 32
_JW = (_T * _K) // _NW
_CH = 64


def _make_dispatch_kernel():
    mesh = plsc.VectorSubcoreMesh(core_axis_name="c", subcore_axis_name="s")

    @functools.partial(
        pl.kernel, mesh=mesh,
        out_type=jax.ShapeDtypeStruct((_P, _D), jnp.float32),
        scratch_types=[
            pltpu.VMEM((_CH,), jnp.int32),
            pltpu.VMEM((_CH, _D), jnp.float32),
            pltpu.SemaphoreType.DMA,
        ],
    )
    def dispatch(pos_hbm, xf_hbm, xs_hbm, idx_v, rows_v, sem):
        wid = lax.axis_index("s") * 2 + lax.axis_index("c")
        t0 = (wid % (_T // _JW)) * _JW
        j0 = wid * _JW
        for cth in range(_JW // _CH):
            pltpu.sync_copy(pos_hbm.at[pl.ds(j0 + cth * _CH, _CH)], idx_v)
            pltpu.sync_copy(xf_hbm.at[pl.ds(t0 + cth * _CH, _CH)], rows_v)
            pltpu.async_copy(rows_v, xs_hbm.at[idx_v], sem).wait()

    return dispatch


_TW = _T // _NW
_CT = 16


def _make_combine_kernel():
    mesh = plsc.VectorSubcoreMesh(core_axis_name="c", subcore_axis_name="s")
    nsteps = (_TW // _CT) * 2

    @functools.partial(
        pl.kernel, mesh=mesh,
        out_type=jax.ShapeDtypeStruct((_T, _D), jnp.float32),
        scratch_types=[
            pltpu.VMEM((_TW,), jnp.int32),
            pltpu.VMEM((_TW,), jnp.int32),
            pltpu.VMEM((_TW, 16), jnp.float32),
            pltpu.VMEM((_TW, 16), jnp.float32),
            pltpu.VMEM((_CT,), jnp.int32),
            pltpu.VMEM((_CT,), jnp.int32),
            pltpu.VMEM((_CT, _D), jnp.float32),
            pltpu.VMEM((_CT, _D), jnp.float32),
            pltpu.VMEM((_CT, _D), jnp.float32),
            pltpu.SemaphoreType.DMA,
            pltpu.SemaphoreType.DMA,
        ],
    )
    def combine(pos_hbm, pb_hbm, y_hbm, out_hbm, pos0_w, pos1_w, pb0_w,
                pb1_w, idx_a, idx_b, rows_a, rows_b, out_c, sem_a, sem_b):
        wid = lax.axis_index("s") * 2 + lax.axis_index("c")
        t0 = wid * _TW
        pltpu.sync_copy(pos_hbm.at[0, pl.ds(t0, _TW)], pos0_w)
        pltpu.sync_copy(pos_hbm.at[1, pl.ds(t0, _TW)], pos1_w)
        pltpu.sync_copy(pb_hbm.at[0, pl.ds(t0, _TW)], pb0_w)
        pltpu.sync_copy(pb_hbm.at[1, pl.ds(t0, _TW)], pb1_w)

        def fire(s):
            c, h = s // 2, s % 2
            posw = pos0_w if h == 0 else pos1_w
            idxv = idx_a if s % 2 == 0 else idx_b
            rowsv = rows_a if s % 2 == 0 else rows_b
            semv = sem_a if s % 2 == 0 else sem_b
            idxv[pl.ds(0, _CT)] = posw[pl.ds(c * _CT, _CT)]
            return pltpu.async_copy(y_hbm.at[idxv], rowsv, semv)

        def compute(s):
            c, h = s // 2, s % 2
            pbw = pb0_w if h == 0 else pb1_w
            rowsv = rows_a if s % 2 == 0 else rows_b
            pbs = [pbw[c * _CT + i, pl.ds(0, 16)] for i in range(_CT)]

            def body(j, acc):
                sl = pl.ds(j * 16, 16)
                for i in range(_CT):
                    if h == 0:
                        out_c[i, sl] = pbs[i] * rowsv[i, sl]
                    else:
                        out_c[i, sl] = out_c[i, sl] + pbs[i] * rowsv[i, sl]
                return acc

            lax.fori_loop(0, _D // 16, body, 0)

        pending = fire(0)
        for s in range(nsteps):
            nxt = fire(s + 1) if s + 1 < nsteps else None
            pending.wait()
            compute(s)
            if s % 2 == 1:
                pltpu.sync_copy(
                    out_c, out_hbm.at[pl.ds(t0 + (s // 2) * _CT, _CT)])
            pending = nxt

    return combine


def _ffn_block_kernel(be_ref, nb_ref, x_ref, w1_ref, b1_ref, w2_ref, b2_ref,
                      o_ref, w1s, w2s, b1s):
    i = pl.program_id(0)
    f = pl.program_id(1)

    @pl.when(i < nb_ref[0])
    def _():
        new_w = (i == 0) | (be_ref[i] != be_ref[jnp.maximum(i - 1, 0)])

        @pl.when(new_w)
        def _():
            w1s[f] = w1_ref[0].astype(jnp.bfloat16)
            w2s[f] = w2_ref[0].astype(jnp.bfloat16)
            b1s[f] = b1_ref[0]

        @pl.when(f == _NF - 1)
        def _():
            xb = x_ref[...].astype(jnp.bfloat16)
            o = b2_ref[0] + jnp.zeros((_R, _D), jnp.float32)
            for ff in range(_NF):
                h = jnp.dot(xb, w1s[ff], preferred_element_type=jnp.float32)
                h = h + b1s[ff]
                h = 0.5 * h * (1.0 + jax.lax.erf(h * 0.7071067811865476))
                o = o + jnp.dot(h.astype(jnp.bfloat16), w2s[ff],
                                preferred_element_type=jnp.float32)
            o_ref[...] = o


def kernel(x, W1, b1, W2, b2, Wr, br):
    bsz, seq, d = x.shape
    xf = x.reshape(-1, d)

    logits = xf @ Wr + br

    pos2, p2, be_row, nb = pl.pallas_call(
        _plan_kernel,
        out_shape=[
            jax.ShapeDtypeStruct((_K, _T), jnp.int32),
            jax.ShapeDtypeStruct((_K, _T), jnp.float32),
            jax.ShapeDtypeStruct((1, 128), jnp.int32),
            jax.ShapeDtypeStruct((1, 1), jnp.int32),
        ],
    )(logits.T)

    block_expert = be_row[0, :_MAXB]
    total_blocks = nb[0]

    pos_flat = pos2.reshape(-1)
    x_sorted = _make_dispatch_kernel()(pos_flat, xf)

    def _wf(i, f, be, nb):
        new_w = (i == 0) | (be[i] != be[jnp.maximum(i - 1, 0)])
        return jnp.where(new_w, f, _NF - 1)

    grid_spec = pltpu.PrefetchScalarGridSpec(
        num_scalar_prefetch=2,
        grid=(_MAXB, _NF),
        in_specs=[
            pl.BlockSpec((_R, _D),
                         lambda i, f, be, nb: (jnp.where(i < nb[0], i, 0), 0)),
            pl.BlockSpec((1, _D, _FH),
                         lambda i, f, be, nb: (be[i], 0, _wf(i, f, be, nb))),
            pl.BlockSpec((1, 1, _FH),
                         lambda i, f, be, nb: (be[i], 0, _wf(i, f, be, nb))),
            pl.BlockSpec((1, _FH, _D),
                         lambda i, f, be, nb: (be[i], _wf(i, f, be, nb), 0)),
            pl.BlockSpec((1, 1, _D), lambda i, f, be, nb: (be[i], 0, 0)),
        ],
        out_specs=pl.BlockSpec((_R, _D), lambda i, f, be, nb: (i, 0)),
        scratch_shapes=[
            pltpu.VMEM((_NF, _D, _FH), jnp.bfloat16),
            pltpu.VMEM((_NF, _FH, _D), jnp.bfloat16),
            pltpu.VMEM((_NF, 1, _FH), jnp.float32),
        ],
    )
    y = pl.pallas_call(
        _ffn_block_kernel,
        grid_spec=grid_spec,
        out_shape=jax.ShapeDtypeStruct((_P, _D), jnp.float32),
    )(block_expert, total_blocks, x_sorted, W1,
      b1.reshape(_E, 1, _F), W2, b2.reshape(_E, 1, _D))

    pbb = jnp.broadcast_to(p2[:, :, None], (_K, _T, 16))
    out = _make_combine_kernel()(pos2, pbb, y)
    return out.reshape(bsz, seq, d)
